# Initial kernel scaffold; baseline (speedup 1.0000x reference)
import jax
import jax.numpy as jnp
from jax.experimental import pallas as pl


def kernel(z_i, z_j):
    raise NotImplementedError("write your pallas kernel here")


# speedup vs baseline: 2.7574x; 2.7574x over previous
import jax
import jax.numpy as jnp
from jax.experimental import pallas as pl
from jax.experimental.pallas import tpu as pltpu

_N = 4096
_D = 128
_BR = 512
_NBLK = _N // _BR
_TEMP = 1.0


def _loss_kernel(zi_ref, zj_ref, out_ref, pos_s, nsum_s, nneg_s):
    i = pl.program_id(0)
    zi = zi_ref[...]
    zj = zj_ref[...]
    sim = jax.lax.dot_general(
        zi, zj, (((1,), (1,)), ((), ())),
        preferred_element_type=jnp.float32) / _TEMP
    pos = jnp.max(sim, axis=1, keepdims=True)
    mask = sim < pos
    neg_sum = jnp.sum(jnp.where(mask, sim, 0.0), axis=1, keepdims=True)
    num_neg = jnp.sum(mask.astype(jnp.float32), axis=1, keepdims=True)

    rows = pl.ds(i * _BR, _BR)
    pos_s[rows, :] = pos
    nsum_s[rows, :] = neg_sum
    nneg_s[rows, :] = num_neg

    @pl.when(i == _NBLK - 1)
    def _finalize():
        pos_a = pos_s[...]
        nsum_a = nsum_s[...]
        nneg_a = nneg_s[...]
        max_neg = jnp.max(nneg_a)
        neg_mean = nsum_a / max_neg
        exp_pos = jnp.exp(pos_a)
        exp_neg = jnp.exp(jnp.minimum(nsum_a, 30.0))
        info_nce = -jnp.mean(jnp.log(exp_pos / exp_neg))
        triplet = jnp.mean(jnp.maximum(pos_a - neg_mean + 1.0, 0.0))
        out_ref[...] = jnp.reshape(info_nce + triplet, (1, 1))


def kernel(z_i, z_j):
    out = pl.pallas_call(
        _loss_kernel,
        grid=(_NBLK,),
        in_specs=[
            pl.BlockSpec((_BR, _D), lambda i: (i, 0)),
            pl.BlockSpec((_N, _D), lambda i: (0, 0)),
        ],
        out_specs=pl.BlockSpec((1, 1), lambda i: (0, 0)),
        out_shape=jax.ShapeDtypeStruct((1, 1), jnp.float32),
        scratch_shapes=[
            pltpu.VMEM((_N, 1), jnp.float32),
            pltpu.VMEM((_N, 1), jnp.float32),
            pltpu.VMEM((_N, 1), jnp.float32),
        ],
    )(z_i, z_j)
    return out[0, 0]


# speedup vs baseline: 3.0045x; 1.0896x over previous
import jax
import jax.numpy as jnp
from jax.experimental import pallas as pl
from jax.experimental.pallas import tpu as pltpu

_N = 4096
_D = 128
_BR = 512
_NBLK = _N // _BR
_TEMP = 1.0


def _loss_kernel(zi_ref, zj_ref, out_ref, pos_s, nneg_s, rsum_s, csum_s):
    i = pl.program_id(0)
    zj = zj_ref[...]
    zi = zi_ref[...]

    @pl.when(i == 0)
    def _colsum():
        csum_s[...] = jnp.sum(zj, axis=0, keepdims=True)

    sim = jax.lax.dot_general(
        zi, zj, (((1,), (1,)), ((), ())),
        preferred_element_type=jnp.float32) / _TEMP
    pos = jnp.max(sim, axis=1, keepdims=True)
    num_neg = jnp.sum((sim < pos).astype(jnp.float32), axis=1, keepdims=True)
    row_sum = jax.lax.dot_general(
        zi, csum_s[...], (((1,), (1,)), ((), ())),
        preferred_element_type=jnp.float32) / _TEMP

    rows = pl.ds(i * _BR, _BR)
    pos_s[rows, :] = pos
    nneg_s[rows, :] = num_neg
    rsum_s[rows, :] = row_sum

    @pl.when(i == _NBLK - 1)
    def _finalize():
        shp = (_N // 128, 128)
        pos_a = jnp.reshape(pos_s[...], shp)
        nneg_a = jnp.reshape(nneg_s[...], shp)
        rsum_a = jnp.reshape(rsum_s[...], shp)
        neg_sum = rsum_a - pos_a * (_N - nneg_a)
        max_neg = jnp.max(nneg_a)
        neg_mean = neg_sum / max_neg
        exp_pos = jnp.exp(pos_a)
        exp_neg = jnp.exp(jnp.minimum(neg_sum, 30.0))
        info_nce = -jnp.mean(jnp.log(exp_pos / exp_neg))
        triplet = jnp.mean(jnp.maximum(pos_a - neg_mean + 1.0, 0.0))
        out_ref[...] = jnp.reshape(info_nce + triplet, (1, 1))


def kernel(z_i, z_j):
    out = pl.pallas_call(
        _loss_kernel,
        grid=(_NBLK,),
        in_specs=[
            pl.BlockSpec((_BR, _D), lambda i: (i, 0)),
            pl.BlockSpec((_N, _D), lambda i: (0, 0)),
        ],
        out_specs=pl.BlockSpec((1, 1), lambda i: (0, 0)),
        out_shape=jax.ShapeDtypeStruct((1, 1), jnp.float32),
        scratch_shapes=[
            pltpu.VMEM((_N, 1), jnp.float32),
            pltpu.VMEM((_N, 1), jnp.float32),
            pltpu.VMEM((_N, 1), jnp.float32),
            pltpu.VMEM((1, _D), jnp.float32),
        ],
    )(z_i, z_j)
    return out[0, 0]
